# Initial kernel scaffold; baseline (speedup 1.0000x reference)
import jax
import jax.numpy as jnp
from jax.experimental import pallas as pl


def kernel(points, W1, g1, b1, W2, g2, b2, W3, g3, b3, W4, g4, b4):
    raise NotImplementedError("write your pallas kernel here")


# speedup vs baseline: 2.0017x; 2.0017x over previous
import functools

import jax
import jax.numpy as jnp
from jax.experimental import pallas as pl
from jax.experimental.pallas import tpu as pltpu

VOXEL_SIZE = 0.1
EPS = 1e-5
S = 512
R_BLK = 2048


def _gelu(x):
    return x * 0.5 * (1.0 + jax.lax.erf(x * 0.7071067811865476))


def _bn_apply(x, stats, g, b, total):
    mu = stats[0:1, :] / total
    ex2 = stats[1:2, :] / total
    var = ex2 - mu * mu
    rstd = jax.lax.rsqrt(var + EPS)
    return (x - mu) * rstd * g + b


def _colstats(x):
    s = jnp.sum(x, axis=0, keepdims=True)
    s2 = jnp.sum(x * x, axis=0, keepdims=True)
    return jnp.concatenate([s, s2], axis=0)


def _k_prep(uh_ref, flat_ref, g4r_ref, w1_ref, idx_ref, norm_ref,
            stats_ref, acc):
    i = pl.program_id(0)
    uh = uh_ref[0, :]
    valid = uh >= 0
    vz = jnp.bitwise_and(uh, S - 1)
    q = jax.lax.shift_right_logical(uh, 9)
    vy = jnp.bitwise_and(q, S - 1)
    q = jax.lax.shift_right_logical(q, 9)
    vx = jnp.bitwise_and(q, S - 1)
    vb = jax.lax.shift_right_logical(q, 9)
    dec = jnp.stack([vb, vx, vy, vz], axis=0)
    idx_ref[...] = jnp.where(valid[None, :], dec, -1)

    g4 = g4r_ref[...]
    cent = g4[:, 0:3] / (g4[:, 3:4] + 1.0)
    norm = flat_ref[...] - cent
    norm_ref[...] = norm
    x1 = jnp.dot(norm, w1_ref[...], preferred_element_type=jnp.float32)

    @pl.when(i == 0)
    def _():
        acc[...] = jnp.zeros_like(acc)

    acc[...] += _colstats(x1)
    stats_ref[...] = acc[...]


def _k_l12(norm_ref, w1_ref, stats1_ref, g1_ref, b1_ref, w2_ref,
           x2_ref, stats_ref, acc, *, total):
    i = pl.program_id(0)
    x1 = jnp.dot(norm_ref[...], w1_ref[...], preferred_element_type=jnp.float32)
    p1 = _gelu(_bn_apply(x1, stats1_ref[...], g1_ref[...], b1_ref[...], total))
    x2 = jnp.dot(p1, w2_ref[...], preferred_element_type=jnp.float32)
    x2_ref[...] = x2

    @pl.when(i == 0)
    def _():
        acc[...] = jnp.zeros_like(acc)

    acc[...] += _colstats(x2)
    stats_ref[...] = acc[...]


def _k_l3(x2_ref, stats2_ref, g2_ref, b2_ref, flat_ref, c4_ref, w3_ref,
          x3_ref, stats_ref, acc, *, total):
    i = pl.program_id(0)
    p2 = _gelu(_bn_apply(x2_ref[...], stats2_ref[...], g2_ref[...],
                         b2_ref[...], total))
    c4 = c4_ref[...]
    pooled = c4[:, 0:3] / (c4[:, 3:4] + 1.0)
    w3 = w3_ref[...]
    x3 = jnp.dot(p2, w3[3:259, :], preferred_element_type=jnp.float32)
    x3 += jnp.dot(flat_ref[...], w3[0:3, :], preferred_element_type=jnp.float32)
    x3 += jnp.dot(pooled, w3[259:262, :], preferred_element_type=jnp.float32)
    x3_ref[...] = x3

    @pl.when(i == 0)
    def _():
        acc[...] = jnp.zeros_like(acc)

    acc[...] += _colstats(x3)
    stats_ref[...] = acc[...]


def _k_l4(x3_ref, stats3_ref, g3_ref, b3_ref, w4_ref, x4_ref, stats_ref,
          acc, *, total):
    i = pl.program_id(0)
    p3 = _gelu(_bn_apply(x3_ref[...], stats3_ref[...], g3_ref[...],
                         b3_ref[...], total))
    x4 = jnp.dot(p3, w4_ref[...], preferred_element_type=jnp.float32)
    x4_ref[...] = x4

    @pl.when(i == 0)
    def _():
        acc[...] = jnp.zeros_like(acc)

    acc[...] += _colstats(x4)
    stats_ref[...] = acc[...]


def _k_final(x4_ref, stats4_ref, g4_ref, b4_ref, out_ref, *, total):
    out_ref[...] = _gelu(_bn_apply(x4_ref[...], stats4_ref[...],
                                   g4_ref[...], b4_ref[...], total))


def _row_spec(c):
    return pl.BlockSpec((R_BLK, c), lambda i: (i, 0))


def _full_spec(r, c):
    return pl.BlockSpec((r, c), lambda i: (0, 0))


def _mlp(uh1, flat, g4r, csum4, W1, g1, b1, W2, g2, b2, W3, g3, b3, W4,
         g4, b4, total):
    nb = total // R_BLK
    D = W2.shape[0]
    f32 = jnp.float32
    stats_sd = jax.ShapeDtypeStruct((2, D), f32)
    x_sd = jax.ShapeDtypeStruct((total, D), f32)
    scr = [pltpu.VMEM((2, D), f32)]
    g1r, b1r = g1[None, :], b1[None, :]
    g2r, b2r = g2[None, :], b2[None, :]
    g3r, b3r = g3[None, :], b3[None, :]
    g4rr, b4rr = g4[None, :], b4[None, :]

    idx, norm, stats1 = pl.pallas_call(
        _k_prep,
        grid=(nb,),
        in_specs=[pl.BlockSpec((1, R_BLK), lambda i: (0, i)), _row_spec(3),
                  _row_spec(4), _full_spec(3, D)],
        out_specs=[pl.BlockSpec((4, R_BLK), lambda i: (0, i)), _row_spec(3),
                   _full_spec(2, D)],
        out_shape=[jax.ShapeDtypeStruct((4, total), jnp.int32),
                   jax.ShapeDtypeStruct((total, 3), f32), stats_sd],
        scratch_shapes=scr,
    )(uh1[None, :], flat, g4r, W1)

    x2, stats2 = pl.pallas_call(
        functools.partial(_k_l12, total=float(total)),
        grid=(nb,),
        in_specs=[_row_spec(3), _full_spec(3, D), _full_spec(2, D),
                  _full_spec(1, D), _full_spec(1, D), _full_spec(D, D)],
        out_specs=[_row_spec(D), _full_spec(2, D)],
        out_shape=[x_sd, stats_sd],
        scratch_shapes=scr,
    )(norm, W1, stats1, g1r, b1r, W2)

    x3, stats3 = pl.pallas_call(
        functools.partial(_k_l3, total=float(total)),
        grid=(nb,),
        in_specs=[_row_spec(D), _full_spec(2, D), _full_spec(1, D),
                  _full_spec(1, D), _row_spec(3), _row_spec(4),
                  _full_spec(D + 6, D)],
        out_specs=[_row_spec(D), _full_spec(2, D)],
        out_shape=[x_sd, stats_sd],
        scratch_shapes=scr,
    )(x2, stats2, g2r, b2r, flat, csum4, W3)

    x4, stats4 = pl.pallas_call(
        functools.partial(_k_l4, total=float(total)),
        grid=(nb,),
        in_specs=[_row_spec(D), _full_spec(2, D), _full_spec(1, D),
                  _full_spec(1, D), _full_spec(D, D)],
        out_specs=[_row_spec(D), _full_spec(2, D)],
        out_shape=[x_sd, stats_sd],
        scratch_shapes=scr,
    )(x3, stats3, g3r, b3r, W4)

    agg = pl.pallas_call(
        functools.partial(_k_final, total=float(total)),
        grid=(nb,),
        in_specs=[_row_spec(D), _full_spec(2, D), _full_spec(1, D),
                  _full_spec(1, D)],
        out_specs=_row_spec(D),
        out_shape=x_sd,
    )(x4, stats4, g4rr, b4rr)

    return idx, norm, agg


def kernel(points, W1, g1, b1, W2, g2, b2, W3, g3, b3, W4, g4, b4):
    B, N, _ = points.shape
    total = B * N
    flat = points.reshape(-1, 3)
    pmin = jnp.min(flat, axis=0)
    vc = jnp.floor((flat - pmin) / VOXEL_SIZE).astype(jnp.int32)
    batch_ids = jnp.repeat(jnp.arange(B, dtype=jnp.int32), N)
    h1 = ((batch_ids * S + vc[:, 0]) * S + vc[:, 1]) * S + vc[:, 2]

    p1 = jnp.argsort(h1)
    s1 = h1[p1]
    f1 = jnp.concatenate([jnp.ones((1,), jnp.int32),
                          (s1[1:] != s1[:-1]).astype(jnp.int32)])
    r1 = jnp.cumsum(f1) - 1
    uh1 = jnp.full((total,), -1, jnp.int32).at[r1].set(s1)

    vc2 = jnp.floor(flat / VOXEL_SIZE).astype(jnp.int32)
    h2 = vc2[:, 0] * 73856093 + vc2[:, 1] * 19349663 + vc2[:, 2] * 83492791
    p2 = jnp.argsort(h2)
    s2 = h2[p2]
    f2 = jnp.concatenate([jnp.ones((1,), jnp.int32),
                          (s2[1:] != s2[:-1]).astype(jnp.int32)])
    r2 = jnp.cumsum(f2) - 1
    pts4 = jnp.concatenate([flat, jnp.ones((total, 1), jnp.float32)], axis=1)
    csum4 = jnp.zeros((total, 4), jnp.float32).at[r2].add(pts4[p2])
    g4r = csum4[p1]

    idx, norm, agg = _mlp(uh1, flat, g4r, csum4, W1, g1, b1, W2, g2, b2,
                          W3, g3, b3, W4, g4, b4, total)
    return idx, agg, norm
